# baseline (speedup 1.0000x reference)
import functools

import jax
import jax.numpy as jnp
from jax import lax
from jax.experimental import pallas as pl
from jax.experimental.pallas import tpu as pltpu
from jax.experimental.pallas import tpu_sc as plsc

VOCAB = 100000
DIM = 768
NTAB = 6
B, S = 4, 2048
NIDS = B * S

NC, NS = 2, 16
NW = NC * NS
IDS_PER_W = NIDS // NW
CH = 64
NCHUNK = IDS_PER_W // CH


def _make_gather():
  mesh = plsc.VectorSubcoreMesh(core_axis_name="c", subcore_axis_name="s")

  @functools.partial(
      pl.kernel,
      out_type=tuple(
          jax.ShapeDtypeStruct((NIDS, DIM), jnp.float32) for _ in range(2 * NTAB)
      ),
      mesh=mesh,
      scratch_types=[
          pltpu.VMEM((NCHUNK, CH), jnp.int32),
          pltpu.VMEM((CH, DIM), jnp.float32),
          pltpu.VMEM((CH, DIM), jnp.float32),
          pltpu.SemaphoreType.DMA,
          pltpu.SemaphoreType.DMA,
          pltpu.SemaphoreType.DMA,
          pltpu.SemaphoreType.DMA,
          pltpu.SemaphoreType.DMA,
          pltpu.SemaphoreType.DMA,
      ],
  )
  def gather12(idx_hbm, w0, w1, w2, w3, w4, w5,
               o0, o1, o2, o3, o4, o5, d0, d1, d2, d3, d4, d5,
               idx_v, rows0, rows1, gs0, gs1, ws0, ws1, ds0, ds1):
    wid = lax.axis_index("s") * NC + lax.axis_index("c")
    base = wid * IDS_PER_W
    pltpu.sync_copy(idx_hbm.at[wid], idx_v)
    bufs = (rows0, rows1)
    gsems = (gs0, gs1)
    wsems = (ws0, ws1)
    dsems = (ds0, ds1)
    work = [
        (w, o, d, c)
        for w, o, d in (
            (w0, o0, d0), (w1, o1, d1), (w2, o2, d2),
            (w3, o3, d3), (w4, o4, d4), (w5, o5, d5),
        )
        for c in range(NCHUNK)
    ]
    n = len(work)
    gathers = [None] * n
    writes = [None] * n
    dups = [None] * n

    def _slice(i):
      return pl.ds(base + work[i][3] * CH, CH)

    w_, _, _, c_ = work[0]
    gathers[0] = pltpu.async_copy(w_.at[idx_v.at[c_]], bufs[0], gsems[0])
    for i in range(n):
      b = i % 2
      nb = (i + 1) % 2
      if i + 1 < n:
        if i >= 1:
          writes[i - 1].wait()
          if i >= 3:
            dups[i - 3].wait()
          _, o, d, _ = work[i - 1]
          sl = _slice(i - 1)
          dups[i - 1] = pltpu.async_copy(o.at[sl], d.at[sl], dsems[nb])
        w, _, _, c = work[i + 1]
        gathers[i + 1] = pltpu.async_copy(w.at[idx_v.at[c]], bufs[nb], gsems[nb])
      gathers[i].wait()
      _, o, d, c = work[i]
      writes[i] = pltpu.async_copy(bufs[b], o.at[_slice(i)], wsems[b])
    for i in (n - 2, n - 1):
      writes[i].wait()
      _, o, d, _ = work[i]
      sl = _slice(i)
      dups[i] = pltpu.async_copy(o.at[sl], d.at[sl], dsems[i % 2])
    for i in range(n - 4, n):
      if i >= 0 and dups[i] is not None:
        dups[i].wait()

  return gather12


_gather12 = _make_gather()


def kernel(inputs, W0, W1, W2, W3, W4, W5):
  idx = inputs.reshape(NW, NCHUNK, CH)
  outs = _gather12(idx, W0, W1, W2, W3, W4, W5)
  ve = tuple(o.reshape(B, S, DIM) for o in outs[:NTAB])
  dup = tuple(o.reshape(B, S, DIM) for o in outs[NTAB:])
  return ve + tuple(reversed(dup))


# speedup vs baseline: 24.5801x; 24.5801x over previous
import functools

import jax
import jax.numpy as jnp
from jax import lax
from jax.experimental import pallas as pl
from jax.experimental.pallas import tpu as pltpu
from jax.experimental.pallas import tpu_sc as plsc

VOCAB = 100000
DIM = 768
NTAB = 6
B, S = 4, 2048
NIDS = B * S

NC, NS = 2, 16
NW = NC * NS
IDS_PER_W = NIDS // NW
CH = 64
NCHUNK = IDS_PER_W // CH


def _make_gather():
  mesh = plsc.VectorSubcoreMesh(core_axis_name="c", subcore_axis_name="s")

  @functools.partial(
      pl.kernel,
      out_type=tuple(
          jax.ShapeDtypeStruct((NIDS, DIM), jnp.float32) for _ in range(2 * NTAB)
      ),
      mesh=mesh,
      scratch_types=[
          pltpu.VMEM((NCHUNK, CH), jnp.int32),
          pltpu.VMEM((CH, DIM), jnp.float32),
          pltpu.VMEM((CH, DIM), jnp.float32),
          pltpu.SemaphoreType.DMA,
          pltpu.SemaphoreType.DMA,
          pltpu.SemaphoreType.DMA,
          pltpu.SemaphoreType.DMA,
      ],
  )
  def gather12(idx_hbm, w0, w1, w2, w3, w4, w5,
               o0, o1, o2, o3, o4, o5, d0, d1, d2, d3, d4, d5,
               idx_v, rows0, rows1, gs0, gs1, ws0, ws1):
    wid = lax.axis_index("s") * NC + lax.axis_index("c")
    base = wid * IDS_PER_W
    pltpu.sync_copy(idx_hbm.at[wid], idx_v)
    bufs = (rows0, rows1)
    gsems = (gs0, gs1)
    wsems = (ws0, ws1)
    work = [
        (w, o, d, c)
        for w, o, d in (
            (w0, o0, d0), (w1, o1, d1), (w2, o2, d2),
            (w3, o3, d3), (w4, o4, d4), (w5, o5, d5),
        )
        for c in range(NCHUNK)
    ]
    n = len(work)
    gathers = [None] * n
    writes = [None] * n
    w_, _, _, c_ = work[0]
    gathers[0] = pltpu.async_copy(w_.at[idx_v.at[c_]], bufs[0], gsems[0])
    for i in range(n):
      b = i % 2
      nb = (i + 1) % 2
      if i + 1 < n:
        if i >= 1:
          for wr in writes[i - 1]:
            wr.wait()
        w, _, _, c = work[i + 1]
        gathers[i + 1] = pltpu.async_copy(w.at[idx_v.at[c]], bufs[nb], gsems[nb])
      gathers[i].wait()
      _, o, d, c = work[i]
      sl = pl.ds(base + c * CH, CH)
      writes[i] = (
          pltpu.async_copy(bufs[b], o.at[sl], wsems[b]),
          pltpu.async_copy(bufs[b], d.at[sl], wsems[b]),
      )
    for wr in writes[n - 2]:
      wr.wait()
    for wr in writes[n - 1]:
      wr.wait()

  return gather12


_gather12 = _make_gather()


def kernel(inputs, W0, W1, W2, W3, W4, W5):
  idx = inputs.reshape(NW, NCHUNK, CH)
  outs = _gather12(idx, W0, W1, W2, W3, W4, W5)
  ve = tuple(o.reshape(B, S, DIM) for o in outs[:NTAB])
  dup = tuple(o.reshape(B, S, DIM) for o in outs[NTAB:])
  return ve + tuple(reversed(dup))
